# baseline (speedup 1.0000x reference)
import functools

import jax
import jax.numpy as jnp
from jax.experimental import pallas as pl

B, T, D, NT, H, DH, FF, E, K = 64, 576, 768, 4, 12, 64, 3072, 8, 2
HQ = H * NT
BB = 4
FFB = 768
HI = jax.lax.Precision.HIGHEST


def _dot(a, b, dims, precision=HI):
    return jax.lax.dot_general(a, b, (dims, ((), ())),
                               precision=precision,
                               preferred_element_type=jnp.float32)


def _prep_kernel(probe_ref, wq_ref, bq_ref, wk_ref, qk_ref):
    q = _dot(probe_ref[:], wq_ref[:], ((1,), (1,))) + bq_ref[:]
    q_tiled = jnp.tile(q, (H, 1))
    row_h = jax.lax.broadcasted_iota(jnp.int32, (HQ, D), 0) // NT
    col_h = jax.lax.broadcasted_iota(jnp.int32, (HQ, D), 1) // DH
    q_block = jnp.where(row_h == col_h, q_tiled, 0.0)
    qk_ref[:] = _dot(q_block, wk_ref[:], ((1,), (0,))) * (1.0 / 8.0)


def _attn_kernel(hs_ref, qk_ref, z_ref):
    hs = hs_ref[:]
    s = _dot(hs.reshape(BB * T, D), qk_ref[:], ((1,), (1,)))
    s3 = s.reshape(BB, T, HQ)
    m = jnp.max(s3, axis=1, keepdims=True)
    p = jnp.exp(s3 - m)
    att = p / jnp.sum(p, axis=1, keepdims=True)
    for j in range(BB):
        z = _dot(att[j], hs[j], ((0,), (0,)))
        z_ref[:, j, :, :] = z.reshape(H, NT, D)


def _head_kernel(z_ref, wv_ref, bv_ref, wo_ref, bo_ref, lng_ref, lnb_ref,
                 rw_ref, rb_ref, fc2b_ref,
                 x_ref, w_ref, base_ref, stats_ref, loss_ref):
    parts = []
    for h in range(H):
        z_h = z_ref[h].reshape(B * NT, D)
        wv_h = wv_ref[h * DH:(h + 1) * DH, :]
        parts.append(_dot(z_h, wv_h, ((1,), (1,))))
    o = jnp.concatenate(parts, axis=1) + bv_ref[:]
    attn_out = _dot(o, wo_ref[:], ((1,), (1,))) + bo_ref[:]
    mu = jnp.mean(attn_out, axis=1, keepdims=True)
    xc = attn_out - mu
    var = jnp.mean(xc * xc, axis=1, keepdims=True)
    xl = xc / jnp.sqrt(var + 1e-6) * lng_ref[:] + lnb_ref[:]
    x_ref[:] = xl
    logits = _dot(xl, rw_ref[:], ((1,), (1,))) + rb_ref[:]
    lm = jnp.max(logits, axis=1, keepdims=True)
    pe = jnp.exp(logits - lm)
    probs = pe / jnp.sum(pe, axis=1, keepdims=True)
    cols = jax.lax.broadcasted_iota(jnp.int32, (B * NT, E), 1)
    m1 = jnp.max(probs, axis=1, keepdims=True)
    i1 = jnp.min(jnp.where(probs == m1, cols, E), axis=1, keepdims=True)
    masked = jnp.where(cols == i1, -1.0, probs)
    m2 = jnp.max(masked, axis=1, keepdims=True)
    i2 = jnp.min(jnp.where(masked == m2, cols, E), axis=1, keepdims=True)
    ssum = m1 + m2
    sel1 = cols == i1
    sel2 = cols == i2
    weights = (jnp.where(sel1, m1 / ssum, 0.0)
               + jnp.where(sel2, m2 / ssum, 0.0))
    w_ref[:] = weights
    disp = sel1.astype(jnp.float32) + sel2.astype(jnp.float32)
    stats = jnp.sum(disp, axis=0, keepdims=True)
    stats_ref[:] = stats
    pmean = jnp.sum(probs, axis=0, keepdims=True) * (1.0 / (B * NT))
    loss_ref[:] = (E / (B * NT)) * jnp.sum(stats * pmean,
                                           axis=1, keepdims=True)
    base_ref[:] = attn_out + _dot(weights, fc2b_ref[:], ((1,), (0,)))


def _ffn_kernel(x_ref, w_ref, base_ref, fc1w_ref, fc1b_ref, fc2w_ref,
                out_ref):
    e = pl.program_id(0)
    fb = pl.program_id(1)

    @pl.when(jnp.logical_and(e == 0, fb == 0))
    def _init():
        out_ref[:] = base_ref[:]

    hpre = _dot(x_ref[:], fc1w_ref[0], ((1,), (1,))) + fc1b_ref[0]
    hact = jax.nn.gelu(hpre, approximate=True)
    part = _dot(hact, fc2w_ref[0], ((1,), (1,)))
    cols = jax.lax.broadcasted_iota(jnp.int32, (B * NT, E), 1)
    w_col = jnp.sum(jnp.where(cols == e, w_ref[:], 0.0), axis=1,
                    keepdims=True)
    out_ref[:] = out_ref[:] + part * w_col


def kernel(hidden_state, probe, in_proj_w, in_proj_b, out_proj_w, out_proj_b,
           ln_g, ln_b, router_w, router_b, fc1_w, fc1_b, fc2_w, fc2_b):
    f32 = jnp.float32
    n = B * NT
    wq, wk, wv = in_proj_w[:D], in_proj_w[D:2 * D], in_proj_w[2 * D:]
    bq = in_proj_b[:D].reshape(1, D)
    bv = in_proj_b[2 * D:].reshape(1, D)

    qk = pl.pallas_call(
        _prep_kernel,
        out_shape=jax.ShapeDtypeStruct((HQ, D), f32),
    )(probe.reshape(NT, D), wq, bq, wk)

    z = pl.pallas_call(
        _attn_kernel,
        grid=(B // BB,),
        in_specs=[
            pl.BlockSpec((BB, T, D), lambda i: (i, 0, 0)),
            pl.BlockSpec((HQ, D), lambda i: (0, 0)),
        ],
        out_specs=pl.BlockSpec((H, BB, NT, D), lambda i: (0, i, 0, 0)),
        out_shape=jax.ShapeDtypeStruct((H, B, NT, D), f32),
    )(hidden_state, qk)

    x, weights, base, stats, loss = pl.pallas_call(
        _head_kernel,
        out_shape=(
            jax.ShapeDtypeStruct((n, D), f32),
            jax.ShapeDtypeStruct((n, E), f32),
            jax.ShapeDtypeStruct((n, D), f32),
            jax.ShapeDtypeStruct((1, E), f32),
            jax.ShapeDtypeStruct((1, 1), f32),
        ),
    )(z, wv, bv, out_proj_w, out_proj_b.reshape(1, D),
      ln_g.reshape(1, D), ln_b.reshape(1, D),
      router_w, router_b.reshape(1, E), fc2_b)

    out = pl.pallas_call(
        _ffn_kernel,
        grid=(E, FF // FFB),
        in_specs=[
            pl.BlockSpec((n, D), lambda e, fb: (0, 0)),
            pl.BlockSpec((n, E), lambda e, fb: (0, 0)),
            pl.BlockSpec((n, D), lambda e, fb: (0, 0)),
            pl.BlockSpec((1, FFB, D), lambda e, fb: (e, fb, 0)),
            pl.BlockSpec((1, 1, FFB), lambda e, fb: (e, 0, fb)),
            pl.BlockSpec((1, D, FFB), lambda e, fb: (e, 0, fb)),
        ],
        out_specs=pl.BlockSpec((n, D), lambda e, fb: (0, 0)),
        out_shape=jax.ShapeDtypeStruct((n, D), f32),
    )(x, weights, base, fc1_w, fc1_b.reshape(E, 1, FF), fc2_w)

    return (out.reshape(B, NT, D), loss.reshape(()), stats.reshape(E))


# speedup vs baseline: 2.0042x; 2.0042x over previous
import functools

import jax
import jax.numpy as jnp
from jax.experimental import pallas as pl

B, T, D, NT, H, DH, FF, E, K = 64, 576, 768, 4, 12, 64, 3072, 8, 2
HQ = H * NT
BB = 4
FFB = 768
HI = jax.lax.Precision.HIGHEST


def _dot(a, b, dims, precision=HI):
    return jax.lax.dot_general(a, b, (dims, ((), ())),
                               precision=precision,
                               preferred_element_type=jnp.float32)


def _split(a):
    hi = a.astype(jnp.bfloat16)
    lo = (a - hi.astype(jnp.float32)).astype(jnp.bfloat16)
    return hi, lo


def _bdot(a, b, dims):
    return jax.lax.dot_general(a, b, (dims, ((), ())),
                               preferred_element_type=jnp.float32)


def _dot3(a, b, dims, a_split=None, b_split=None):
    ah, al = _split(a) if a_split is None else a_split
    bh, bl = _split(b) if b_split is None else b_split
    return _bdot(ah, bh, dims) + (_bdot(ah, bl, dims) + _bdot(al, bh, dims))


def _prep_kernel(probe_ref, wq_ref, bq_ref, wk_ref, qk_ref):
    q = _dot(probe_ref[:], wq_ref[:], ((1,), (1,))) + bq_ref[:]
    q_tiled = jnp.tile(q, (H, 1))
    row_h = jax.lax.broadcasted_iota(jnp.int32, (HQ, D), 0) // NT
    col_h = jax.lax.broadcasted_iota(jnp.int32, (HQ, D), 1) // DH
    q_block = jnp.where(row_h == col_h, q_tiled, 0.0)
    qk_ref[:] = _dot(q_block, wk_ref[:], ((1,), (0,))) * (1.0 / 8.0)


def _attn_kernel(hs_ref, qk_ref, z_ref):
    hs = hs_ref[:]
    hs_h, hs_l = _split(hs)
    s = _dot3(None, qk_ref[:], ((1,), (1,)),
              a_split=(hs_h.reshape(BB * T, D),
                       hs_l.reshape(BB * T, D)))
    s3 = s.reshape(BB, T, HQ)
    m = jnp.max(s3, axis=1, keepdims=True)
    p = jnp.exp(s3 - m)
    att = p / jnp.sum(p, axis=1, keepdims=True)
    for j in range(BB):
        z = _dot3(att[j], hs[j], ((0,), (0,)),
                  b_split=(hs_h[j], hs_l[j]))
        z_ref[:, j, :, :] = z.reshape(H, NT, D)


def _head_kernel(z_ref, wv_ref, bv_ref, wo_ref, bo_ref, lng_ref, lnb_ref,
                 rw_ref, rb_ref, fc2b_ref,
                 x_ref, w_ref, base_ref, stats_ref, loss_ref):
    wv_h, wv_l = _split(wv_ref[:])
    parts = []
    for h in range(H):
        z_h = z_ref[h].reshape(B * NT, D)
        sl = slice(h * DH, (h + 1) * DH)
        parts.append(_dot3(z_h, None, ((1,), (1,)),
                           b_split=(wv_h[sl], wv_l[sl])))
    o = jnp.concatenate(parts, axis=1) + bv_ref[:]
    attn_out = _dot3(o, wo_ref[:], ((1,), (1,))) + bo_ref[:]
    mu = jnp.mean(attn_out, axis=1, keepdims=True)
    xc = attn_out - mu
    var = jnp.mean(xc * xc, axis=1, keepdims=True)
    xl = xc / jnp.sqrt(var + 1e-6) * lng_ref[:] + lnb_ref[:]
    x_ref[:] = xl
    logits = _dot3(xl, rw_ref[:], ((1,), (1,))) + rb_ref[:]
    lm = jnp.max(logits, axis=1, keepdims=True)
    pe = jnp.exp(logits - lm)
    probs = pe / jnp.sum(pe, axis=1, keepdims=True)
    cols = jax.lax.broadcasted_iota(jnp.int32, (B * NT, E), 1)
    m1 = jnp.max(probs, axis=1, keepdims=True)
    i1 = jnp.min(jnp.where(probs == m1, cols, E), axis=1, keepdims=True)
    masked = jnp.where(cols == i1, -1.0, probs)
    m2 = jnp.max(masked, axis=1, keepdims=True)
    i2 = jnp.min(jnp.where(masked == m2, cols, E), axis=1, keepdims=True)
    ssum = m1 + m2
    sel1 = cols == i1
    sel2 = cols == i2
    weights = (jnp.where(sel1, m1 / ssum, 0.0)
               + jnp.where(sel2, m2 / ssum, 0.0))
    w_ref[:] = weights
    disp = sel1.astype(jnp.float32) + sel2.astype(jnp.float32)
    stats = jnp.sum(disp, axis=0, keepdims=True)
    stats_ref[:] = stats
    pmean = jnp.sum(probs, axis=0, keepdims=True) * (1.0 / (B * NT))
    loss_ref[:] = (E / (B * NT)) * jnp.sum(stats * pmean,
                                           axis=1, keepdims=True)
    base_ref[:] = attn_out + _dot(weights, fc2b_ref[:], ((1,), (0,)))


def _ffn_kernel(x_ref, w_ref, base_ref, fc1w_ref, fc1b_ref, fc2w_ref,
                out_ref):
    e = pl.program_id(0)
    fb = pl.program_id(1)

    @pl.when(jnp.logical_and(e == 0, fb == 0))
    def _init():
        out_ref[:] = base_ref[:]

    hpre = _bdot(x_ref[:].astype(jnp.bfloat16),
                 fc1w_ref[0].astype(jnp.bfloat16),
                 ((1,), (1,))) + fc1b_ref[0]
    hact = jax.nn.gelu(hpre, approximate=True)
    part = _bdot(hact.astype(jnp.bfloat16),
                 fc2w_ref[0].astype(jnp.bfloat16),
                 ((1,), (1,)))
    cols = jax.lax.broadcasted_iota(jnp.int32, (B * NT, E), 1)
    w_col = jnp.sum(jnp.where(cols == e, w_ref[:], 0.0), axis=1,
                    keepdims=True)
    out_ref[:] = out_ref[:] + part * w_col


def kernel(hidden_state, probe, in_proj_w, in_proj_b, out_proj_w, out_proj_b,
           ln_g, ln_b, router_w, router_b, fc1_w, fc1_b, fc2_w, fc2_b):
    f32 = jnp.float32
    n = B * NT
    wq, wk, wv = in_proj_w[:D], in_proj_w[D:2 * D], in_proj_w[2 * D:]
    bq = in_proj_b[:D].reshape(1, D)
    bv = in_proj_b[2 * D:].reshape(1, D)

    qk = pl.pallas_call(
        _prep_kernel,
        out_shape=jax.ShapeDtypeStruct((HQ, D), f32),
    )(probe.reshape(NT, D), wq, bq, wk)

    z = pl.pallas_call(
        _attn_kernel,
        grid=(B // BB,),
        in_specs=[
            pl.BlockSpec((BB, T, D), lambda i: (i, 0, 0)),
            pl.BlockSpec((HQ, D), lambda i: (0, 0)),
        ],
        out_specs=pl.BlockSpec((H, BB, NT, D), lambda i: (0, i, 0, 0)),
        out_shape=jax.ShapeDtypeStruct((H, B, NT, D), f32),
    )(hidden_state, qk)

    x, weights, base, stats, loss = pl.pallas_call(
        _head_kernel,
        out_shape=(
            jax.ShapeDtypeStruct((n, D), f32),
            jax.ShapeDtypeStruct((n, E), f32),
            jax.ShapeDtypeStruct((n, D), f32),
            jax.ShapeDtypeStruct((1, E), f32),
            jax.ShapeDtypeStruct((1, 1), f32),
        ),
    )(z, wv, bv, out_proj_w, out_proj_b.reshape(1, D),
      ln_g.reshape(1, D), ln_b.reshape(1, D),
      router_w, router_b.reshape(1, E), fc2_b)

    out = pl.pallas_call(
        _ffn_kernel,
        grid=(E, FF // FFB),
        in_specs=[
            pl.BlockSpec((n, D), lambda e, fb: (0, 0)),
            pl.BlockSpec((n, E), lambda e, fb: (0, 0)),
            pl.BlockSpec((n, D), lambda e, fb: (0, 0)),
            pl.BlockSpec((1, FFB, D), lambda e, fb: (e, fb, 0)),
            pl.BlockSpec((1, 1, FFB), lambda e, fb: (e, 0, fb)),
            pl.BlockSpec((1, D, FFB), lambda e, fb: (e, 0, fb)),
        ],
        out_specs=pl.BlockSpec((n, D), lambda e, fb: (0, 0)),
        out_shape=jax.ShapeDtypeStruct((n, D), f32),
    )(x, weights, base, fc1_w, fc1_b.reshape(E, 1, FF), fc2_w)

    return (out.reshape(B, NT, D), loss.reshape(()), stats.reshape(E))


# speedup vs baseline: 2.3429x; 1.1690x over previous
import jax
import jax.numpy as jnp
from jax.experimental import pallas as pl
from jax.experimental.pallas import tpu as pltpu

B, T, D, NT, H, DH, FF, E, K = 64, 576, 768, 4, 12, 64, 3072, 8, 2
HQ = H * NT
BB = 4
FFB = 1536
HI = jax.lax.Precision.HIGHEST


def _dot(a, b, dims, precision=HI):
    return jax.lax.dot_general(a, b, (dims, ((), ())),
                               precision=precision,
                               preferred_element_type=jnp.float32)


def _split(a):
    hi = a.astype(jnp.bfloat16)
    lo = (a - hi.astype(jnp.float32)).astype(jnp.bfloat16)
    return hi, lo


def _bdot(a, b, dims):
    return jax.lax.dot_general(a, b, (dims, ((), ())),
                               preferred_element_type=jnp.float32)


def _dot3(a, b, dims, a_split=None, b_split=None):
    ah, al = _split(a) if a_split is None else a_split
    bh, bl = _split(b) if b_split is None else b_split
    return _bdot(ah, bh, dims) + (_bdot(ah, bl, dims) + _bdot(al, bh, dims))


def _attn_kernel(hs_ref, probe_ref, wq_ref, bq_ref, wk_ref, z_ref, qk_s):
    @pl.when(pl.program_id(0) == 0)
    def _prep():
        q = _dot(probe_ref[:], wq_ref[:], ((1,), (1,))) + bq_ref[0]
        q_tiled = jnp.tile(q, (H, 1))
        row_h = jax.lax.broadcasted_iota(jnp.int32, (HQ, D), 0) // NT
        col_h = jax.lax.broadcasted_iota(jnp.int32, (HQ, D), 1) // DH
        q_block = jnp.where(row_h == col_h, q_tiled, 0.0)
        qk_s[:] = _dot(q_block, wk_ref[:], ((1,), (0,))) * (1.0 / 8.0)

    qk_h, qk_l = _split(qk_s[:])
    hs = hs_ref[:]
    hs_h, hs_l = _split(hs)
    hs_stack = jnp.concatenate([hs_h.reshape(BB * T, D),
                                hs_l.reshape(BB * T, D)], axis=0)
    qk_stack = jnp.concatenate([qk_h, qk_l], axis=0)
    sb = _bdot(hs_stack, qk_stack, ((1,), (1,)))
    s = sb[:BB * T, :HQ] + (sb[:BB * T, HQ:] + sb[BB * T:, :HQ])
    s3 = s.reshape(BB, T, HQ)
    m = jnp.max(s3, axis=1, keepdims=True)
    p = jnp.exp(s3 - m)
    att = p / jnp.sum(p, axis=1, keepdims=True)
    for j in range(BB):
        z = _dot3(att[j], None, ((0,), (0,)),
                  b_split=(hs_h[j], hs_l[j]))
        z_ref[:, j, :, :] = z.reshape(H, NT, D)


def _head_kernel(z_ref, wv_ref, bv_ref, wo_ref, bo_ref, lng_ref, lnb_ref,
                 rw_ref, rb_ref, fc2b_ref,
                 x_ref, w_ref, base_ref, stats_ref, loss_ref):
    wv_h, wv_l = _split(wv_ref[:])
    parts = []
    for h in range(H):
        z_h = z_ref[h].reshape(B * NT, D)
        sl = slice(h * DH, (h + 1) * DH)
        parts.append(_dot3(z_h, None, ((1,), (1,)),
                           b_split=(wv_h[sl], wv_l[sl])))
    o = jnp.concatenate(parts, axis=1) + bv_ref[0]
    attn_out = _dot3(o, wo_ref[:], ((1,), (1,))) + bo_ref[:]
    mu = jnp.mean(attn_out, axis=1, keepdims=True)
    xc = attn_out - mu
    var = jnp.mean(xc * xc, axis=1, keepdims=True)
    xl = xc / jnp.sqrt(var + 1e-6) * lng_ref[:] + lnb_ref[:]
    x_ref[:] = xl
    logits = _dot3(xl, rw_ref[:], ((1,), (1,))) + rb_ref[:]
    lm = jnp.max(logits, axis=1, keepdims=True)
    pe = jnp.exp(logits - lm)
    probs = pe / jnp.sum(pe, axis=1, keepdims=True)
    cols = jax.lax.broadcasted_iota(jnp.int32, (B * NT, E), 1)
    m1 = jnp.max(probs, axis=1, keepdims=True)
    i1 = jnp.min(jnp.where(probs == m1, cols, E), axis=1, keepdims=True)
    masked = jnp.where(cols == i1, -1.0, probs)
    m2 = jnp.max(masked, axis=1, keepdims=True)
    i2 = jnp.min(jnp.where(masked == m2, cols, E), axis=1, keepdims=True)
    ssum = m1 + m2
    sel1 = cols == i1
    sel2 = cols == i2
    weights = (jnp.where(sel1, m1 / ssum, 0.0)
               + jnp.where(sel2, m2 / ssum, 0.0))
    w_ref[:] = weights
    disp = sel1.astype(jnp.float32) + sel2.astype(jnp.float32)
    stats = jnp.sum(disp, axis=0, keepdims=True)
    stats_ref[:] = stats
    pmean = jnp.sum(probs, axis=0, keepdims=True) * (1.0 / (B * NT))
    loss_ref[:] = (E / (B * NT)) * jnp.sum(stats * pmean,
                                           axis=1, keepdims=True)
    base_ref[:] = attn_out + _dot(weights, fc2b_ref[:], ((1,), (0,)))


def _ffn_kernel(x_ref, w_ref, base_ref, fc1w_ref, fc1b_ref, fc2w_ref,
                out_ref):
    e = pl.program_id(0)
    fb = pl.program_id(1)

    @pl.when(jnp.logical_and(e == 0, fb == 0))
    def _init():
        out_ref[:] = base_ref[:]

    hpre = _bdot(x_ref[:].astype(jnp.bfloat16),
                 fc1w_ref[0].astype(jnp.bfloat16),
                 ((1,), (1,))) + fc1b_ref[0]
    hact = jax.nn.gelu(hpre, approximate=True)
    part = _bdot(hact.astype(jnp.bfloat16),
                 fc2w_ref[0].astype(jnp.bfloat16),
                 ((1,), (1,)))
    cols = jax.lax.broadcasted_iota(jnp.int32, (B * NT, E), 1)
    w_col = jnp.sum(jnp.where(cols == e, w_ref[:], 0.0), axis=1,
                    keepdims=True)
    out_ref[:] = out_ref[:] + part * w_col


def kernel(hidden_state, probe, in_proj_w, in_proj_b, out_proj_w, out_proj_b,
           ln_g, ln_b, router_w, router_b, fc1_w, fc1_b, fc2_w, fc2_b):
    f32 = jnp.float32
    n = B * NT
    in_proj_b3 = in_proj_b.reshape(3, 1, D)

    z = pl.pallas_call(
        _attn_kernel,
        grid=(B // BB,),
        in_specs=[
            pl.BlockSpec((BB, T, D), lambda i: (i, 0, 0)),
            pl.BlockSpec((NT, D), lambda i: (0, 0)),
            pl.BlockSpec((D, D), lambda i: (0, 0)),
            pl.BlockSpec((1, 1, D), lambda i: (0, 0, 0)),
            pl.BlockSpec((D, D), lambda i: (1, 0)),
        ],
        out_specs=pl.BlockSpec((H, BB, NT, D), lambda i: (0, i, 0, 0)),
        out_shape=jax.ShapeDtypeStruct((H, B, NT, D), f32),
        scratch_shapes=[pltpu.VMEM((HQ, D), f32)],
    )(hidden_state, probe.reshape(NT, D), in_proj_w, in_proj_b3, in_proj_w)

    x, weights, base, stats, loss = pl.pallas_call(
        _head_kernel,
        grid=(1,),
        in_specs=[
            pl.BlockSpec((H, B, NT, D), lambda i: (0, 0, 0, 0)),
            pl.BlockSpec((D, D), lambda i: (2, 0)),
            pl.BlockSpec((1, 1, D), lambda i: (2, 0, 0)),
            pl.BlockSpec((D, D), lambda i: (0, 0)),
            pl.BlockSpec((1, D), lambda i: (0, 0)),
            pl.BlockSpec((1, D), lambda i: (0, 0)),
            pl.BlockSpec((1, D), lambda i: (0, 0)),
            pl.BlockSpec((E, D), lambda i: (0, 0)),
            pl.BlockSpec((1, E), lambda i: (0, 0)),
            pl.BlockSpec((E, D), lambda i: (0, 0)),
        ],
        out_specs=(
            pl.BlockSpec((n, D), lambda i: (0, 0)),
            pl.BlockSpec((n, E), lambda i: (0, 0)),
            pl.BlockSpec((n, D), lambda i: (0, 0)),
            pl.BlockSpec((1, E), lambda i: (0, 0)),
            pl.BlockSpec((1, 1), lambda i: (0, 0)),
        ),
        out_shape=(
            jax.ShapeDtypeStruct((n, D), f32),
            jax.ShapeDtypeStruct((n, E), f32),
            jax.ShapeDtypeStruct((n, D), f32),
            jax.ShapeDtypeStruct((1, E), f32),
            jax.ShapeDtypeStruct((1, 1), f32),
        ),
    )(z, in_proj_w, in_proj_b3, out_proj_w, out_proj_b.reshape(1, D),
      ln_g.reshape(1, D), ln_b.reshape(1, D),
      router_w, router_b.reshape(1, E), fc2_b)

    out = pl.pallas_call(
        _ffn_kernel,
        grid=(E, FF // FFB),
        in_specs=[
            pl.BlockSpec((n, D), lambda e, fb: (0, 0)),
            pl.BlockSpec((n, E), lambda e, fb: (0, 0)),
            pl.BlockSpec((n, D), lambda e, fb: (0, 0)),
            pl.BlockSpec((1, FFB, D), lambda e, fb: (e, fb, 0)),
            pl.BlockSpec((1, 1, FFB), lambda e, fb: (e, 0, fb)),
            pl.BlockSpec((1, D, FFB), lambda e, fb: (e, 0, fb)),
        ],
        out_specs=pl.BlockSpec((n, D), lambda e, fb: (0, 0)),
        out_shape=jax.ShapeDtypeStruct((n, D), f32),
    )(x, weights, base, fc1_w, fc1_b.reshape(E, 1, FF), fc2_w)

    return (out.reshape(B, NT, D), loss.reshape(()), stats.reshape(E))


# speedup vs baseline: 2.9549x; 1.2612x over previous
import jax
import jax.numpy as jnp
from jax.experimental import pallas as pl
from jax.experimental.pallas import tpu as pltpu

B, T, D, NT, H, DH, FF, E, K = 64, 576, 768, 4, 12, 64, 3072, 8, 2
HQ = H * NT
BB = 4
FFB = 1536
HI = jax.lax.Precision.HIGHEST


def _dot(a, b, dims, precision=HI):
    return jax.lax.dot_general(a, b, (dims, ((), ())),
                               precision=precision,
                               preferred_element_type=jnp.float32)


def _split(a):
    hi = a.astype(jnp.bfloat16)
    lo = (a - hi.astype(jnp.float32)).astype(jnp.bfloat16)
    return hi, lo


def _bdot(a, b, dims):
    return jax.lax.dot_general(a, b, (dims, ((), ())),
                               preferred_element_type=jnp.float32)


def _dot3(a, b, dims, a_split=None, b_split=None):
    ah, al = _split(a) if a_split is None else a_split
    bh, bl = _split(b) if b_split is None else b_split
    return _bdot(ah, bh, dims) + (_bdot(ah, bl, dims) + _bdot(al, bh, dims))


def _dot3n(a_split, b_split, nstack=True):
    ah, al = a_split
    bh, bl = b_split
    dims = ((1,), (1,))
    if not nstack:
        return _bdot(ah, bh, dims) + (_bdot(ah, bl, dims) + _bdot(al, bh, dims))
    n = bh.shape[0]
    p = _bdot(ah, jnp.concatenate([bh, bl], axis=0), dims)
    return (p[:, :n] + p[:, n:]) + _bdot(al, bh, dims)


def _attn_kernel(hs_ref, probe_ref, wq_ref, bq_ref, wk_ref, z_ref, qk_s):
    @pl.when(pl.program_id(0) == 0)
    def _prep():
        q = _dot(probe_ref[:], wq_ref[:], ((1,), (1,))) + bq_ref[0]
        q_tiled = jnp.tile(q, (H, 1))
        row_h = jax.lax.broadcasted_iota(jnp.int32, (HQ, D), 0) // NT
        col_h = jax.lax.broadcasted_iota(jnp.int32, (HQ, D), 1) // DH
        q_block = jnp.where(row_h == col_h, q_tiled, 0.0)
        qk_s[:] = _dot(q_block, wk_ref[:], ((1,), (0,))) * (1.0 / 8.0)

    qk_h, qk_l = _split(qk_s[:])
    hs = hs_ref[:]
    hs_h, hs_l = _split(hs)
    s = _dot3n((hs_h.reshape(BB * T, D), hs_l.reshape(BB * T, D)),
               (qk_h, qk_l))
    s3 = s.reshape(BB, T, HQ)
    m = jnp.max(s3, axis=1, keepdims=True)
    p = jnp.exp(s3 - m)
    att = p / jnp.sum(p, axis=1, keepdims=True)
    for j in range(BB):
        z = _dot3(att[j], None, ((0,), (0,)),
                  b_split=(hs_h[j], hs_l[j]))
        z_ref[:, j, :, :] = z.reshape(H, NT, D)


def _head_kernel(z_ref, wv_ref, bv_ref, wo_ref, bo_ref, lng_ref, lnb_ref,
                 rw_ref, rb_ref, fc2b_ref,
                 x_ref, w_ref, base_ref, stats_ref, loss_ref):
    wv_h, wv_l = _split(wv_ref[:])
    wv_pair = jnp.concatenate([wv_h.reshape(H, DH, D),
                               wv_l.reshape(H, DH, D)], axis=1)
    parts = []
    for h in range(H):
        zh_hi, zh_lo = _split(z_ref[h].reshape(B * NT, D))
        p = _bdot(zh_hi, wv_pair[h], ((1,), (1,)))
        parts.append((p[:, :DH] + p[:, DH:])
                     + _bdot(zh_lo, wv_h[h * DH:(h + 1) * DH], ((1,), (1,))))
    o = jnp.concatenate(parts, axis=1) + bv_ref[0]
    attn_out = _dot3(o, wo_ref[:], ((1,), (1,))) + bo_ref[:]
    mu = jnp.mean(attn_out, axis=1, keepdims=True)
    xc = attn_out - mu
    var = jnp.mean(xc * xc, axis=1, keepdims=True)
    xl = xc / jnp.sqrt(var + 1e-6) * lng_ref[:] + lnb_ref[:]
    x_ref[:] = xl
    logits = _dot3n(_split(xl), _split(rw_ref[:])) + rb_ref[:]
    lm = jnp.max(logits, axis=1, keepdims=True)
    pe = jnp.exp(logits - lm)
    probs = pe / jnp.sum(pe, axis=1, keepdims=True)
    cols = jax.lax.broadcasted_iota(jnp.int32, (B * NT, E), 1)
    m1 = jnp.max(probs, axis=1, keepdims=True)
    i1 = jnp.min(jnp.where(probs == m1, cols, E), axis=1, keepdims=True)
    masked = jnp.where(cols == i1, -1.0, probs)
    m2 = jnp.max(masked, axis=1, keepdims=True)
    i2 = jnp.min(jnp.where(masked == m2, cols, E), axis=1, keepdims=True)
    ssum = m1 + m2
    sel1 = cols == i1
    sel2 = cols == i2
    weights = (jnp.where(sel1, m1 / ssum, 0.0)
               + jnp.where(sel2, m2 / ssum, 0.0))
    w_ref[:] = weights
    disp = sel1.astype(jnp.float32) + sel2.astype(jnp.float32)
    stats = jnp.sum(disp, axis=0, keepdims=True)
    stats_ref[:] = stats
    pmean = jnp.sum(probs, axis=0, keepdims=True) * (1.0 / (B * NT))
    loss_ref[:] = (E / (B * NT)) * jnp.sum(stats * pmean,
                                           axis=1, keepdims=True)
    base_ref[:] = attn_out + _dot(weights, fc2b_ref[:], ((1,), (0,)))


def _ffn_kernel(x_ref, w_ref, base_ref, fc1w_ref, fc1b_ref, fc2w_ref,
                out_ref):
    e = pl.program_id(0)
    fb = pl.program_id(1)

    @pl.when(jnp.logical_and(e == 0, fb == 0))
    def _init():
        out_ref[:] = base_ref[:]

    hpre = _bdot(x_ref[:].astype(jnp.bfloat16),
                 fc1w_ref[0].astype(jnp.bfloat16),
                 ((1,), (1,))) + fc1b_ref[0]
    hact = jax.nn.gelu(hpre, approximate=True)
    part = _bdot(hact.astype(jnp.bfloat16),
                 fc2w_ref[0].astype(jnp.bfloat16),
                 ((1,), (1,)))
    cols = jax.lax.broadcasted_iota(jnp.int32, (B * NT, E), 1)
    w_col = jnp.sum(jnp.where(cols == e, w_ref[:], 0.0), axis=1,
                    keepdims=True)
    out_ref[:] = out_ref[:] + part * w_col


def kernel(hidden_state, probe, in_proj_w, in_proj_b, out_proj_w, out_proj_b,
           ln_g, ln_b, router_w, router_b, fc1_w, fc1_b, fc2_w, fc2_b):
    f32 = jnp.float32
    n = B * NT
    in_proj_b3 = in_proj_b.reshape(3, 1, D)

    z = pl.pallas_call(
        _attn_kernel,
        grid=(B // BB,),
        in_specs=[
            pl.BlockSpec((BB, T, D), lambda i: (i, 0, 0)),
            pl.BlockSpec((NT, D), lambda i: (0, 0)),
            pl.BlockSpec((D, D), lambda i: (0, 0)),
            pl.BlockSpec((1, 1, D), lambda i: (0, 0, 0)),
            pl.BlockSpec((D, D), lambda i: (1, 0)),
        ],
        out_specs=pl.BlockSpec((H, BB, NT, D), lambda i: (0, i, 0, 0)),
        out_shape=jax.ShapeDtypeStruct((H, B, NT, D), f32),
        scratch_shapes=[pltpu.VMEM((HQ, D), f32)],
    )(hidden_state, probe.reshape(NT, D), in_proj_w, in_proj_b3, in_proj_w)

    x, weights, base, stats, loss = pl.pallas_call(
        _head_kernel,
        grid=(1,),
        in_specs=[
            pl.BlockSpec((H, B, NT, D), lambda i: (0, 0, 0, 0)),
            pl.BlockSpec((D, D), lambda i: (2, 0)),
            pl.BlockSpec((1, 1, D), lambda i: (2, 0, 0)),
            pl.BlockSpec((D, D), lambda i: (0, 0)),
            pl.BlockSpec((1, D), lambda i: (0, 0)),
            pl.BlockSpec((1, D), lambda i: (0, 0)),
            pl.BlockSpec((1, D), lambda i: (0, 0)),
            pl.BlockSpec((E, D), lambda i: (0, 0)),
            pl.BlockSpec((1, E), lambda i: (0, 0)),
            pl.BlockSpec((E, D), lambda i: (0, 0)),
        ],
        out_specs=(
            pl.BlockSpec((n, D), lambda i: (0, 0)),
            pl.BlockSpec((n, E), lambda i: (0, 0)),
            pl.BlockSpec((n, D), lambda i: (0, 0)),
            pl.BlockSpec((1, E), lambda i: (0, 0)),
            pl.BlockSpec((1, 1), lambda i: (0, 0)),
        ),
        out_shape=(
            jax.ShapeDtypeStruct((n, D), f32),
            jax.ShapeDtypeStruct((n, E), f32),
            jax.ShapeDtypeStruct((n, D), f32),
            jax.ShapeDtypeStruct((1, E), f32),
            jax.ShapeDtypeStruct((1, 1), f32),
        ),
    )(z, in_proj_w, in_proj_b3, out_proj_w, out_proj_b.reshape(1, D),
      ln_g.reshape(1, D), ln_b.reshape(1, D),
      router_w, router_b.reshape(1, E), fc2_b)

    out = pl.pallas_call(
        _ffn_kernel,
        grid=(E, FF // FFB),
        in_specs=[
            pl.BlockSpec((n, D), lambda e, fb: (0, 0)),
            pl.BlockSpec((n, E), lambda e, fb: (0, 0)),
            pl.BlockSpec((n, D), lambda e, fb: (0, 0)),
            pl.BlockSpec((1, FFB, D), lambda e, fb: (e, fb, 0)),
            pl.BlockSpec((1, 1, FFB), lambda e, fb: (e, 0, fb)),
            pl.BlockSpec((1, D, FFB), lambda e, fb: (e, 0, fb)),
        ],
        out_specs=pl.BlockSpec((n, D), lambda e, fb: (0, 0)),
        out_shape=jax.ShapeDtypeStruct((n, D), f32),
    )(x, weights, base, fc1_w, fc1_b.reshape(E, 1, FF), fc2_w)

    return (out.reshape(B, NT, D), loss.reshape(()), stats.reshape(E))
